# Initial kernel scaffold; baseline (speedup 1.0000x reference)
import jax
import jax.numpy as jnp
from jax.experimental import pallas as pl


def kernel(input_ids, table, gamma, beta):
    raise NotImplementedError("write your pallas kernel here")


# speedup vs baseline: 3.3739x; 3.3739x over previous
import functools

import jax
import jax.numpy as jnp
from jax import lax
from jax.experimental import pallas as pl
from jax.experimental.pallas import tpu as pltpu
from jax.experimental.pallas import tpu_sc as plsc

B, Lseq, V, D = 16384, 200, 100, 10
DP = 16
NB = B - 2
N = NB * Lseq
NW = 32
LANES = 16

CH = 2048
GP = CH // LANES
PW = 102400
NFULL = PW // CH - 1
TAIL_GROUPS = (N - 31 * PW - NFULL * CH) // LANES


def _normalize_table(table, gamma, beta):
    tpad = jnp.zeros((V, DP), jnp.float32).at[:, :D].set(table)
    gpad = jnp.zeros((1, DP), jnp.float32).at[0, :D].set(gamma)
    bpad = jnp.zeros((1, DP), jnp.float32).at[0, :D].set(beta)

    def body(t_ref, g_ref, b_ref, o_ref):
        x = t_ref[...]
        mean = jnp.sum(x, axis=-1, keepdims=True) * (1.0 / D)
        mask = lax.broadcasted_iota(jnp.int32, (V, DP), 1) < D
        cen = jnp.where(mask, x - mean, 0.0)
        var = jnp.sum(cen * cen, axis=-1, keepdims=True) * (1.0 / D)
        r = lax.rsqrt(var + 1e-5)
        o_ref[...] = cen * r * g_ref[...] + b_ref[...]

    return pl.pallas_call(
        body,
        out_shape=jax.ShapeDtypeStruct((V, DP), jnp.float32),
    )(tpad, gpad, bpad)


def _make_gather_kernel():
    mesh = plsc.VectorSubcoreMesh(core_axis_name="c", subcore_axis_name="s")

    @functools.partial(
        pl.kernel,
        out_type=jax.ShapeDtypeStruct((N * D,), jnp.float32),
        mesh=mesh,
        compiler_params=pltpu.CompilerParams(needs_layout_passes=False),
        scratch_types=[
            pltpu.VMEM((V * DP,), jnp.float32),
            pltpu.VMEM((CH,), jnp.int32),
            pltpu.VMEM((CH * D,), jnp.float32),
        ],
    )
    def gather_k(nt_hbm, ids_hbm, out_hbm, nt_v, ids_v, out_v):
        wid = lax.axis_index("s") * 2 + lax.axis_index("c")
        pltpu.sync_copy(nt_hbm, nt_v)
        base = Lseq + wid * PW
        iota10 = lax.iota(jnp.int32, LANES) * D

        def do_chunk(start_pos, ngroups):
            npos = ngroups * LANES
            pltpu.sync_copy(
                ids_hbm.at[pl.ds(start_pos, npos)],
                ids_v.at[pl.ds(0, npos)],
            )

            def group(g, _):
                idsg = ids_v[pl.ds(g * LANES, LANES)]
                row = idsg * DP
                dst0 = g * (LANES * D) + iota10
                for f in range(D):
                    vals = plsc.load_gather(nt_v, [row + f])
                    plsc.store_scatter(out_v, [dst0 + f], vals)
                return 0

            lax.fori_loop(0, ngroups, group, 0, unroll=2)
            pltpu.sync_copy(
                out_v.at[pl.ds(0, npos * D)],
                out_hbm.at[pl.ds((start_pos - Lseq) * D, npos * D)],
            )

        def chunk_body(c, _):
            do_chunk(base + c * CH, GP)
            return 0

        lax.fori_loop(0, NFULL, chunk_body, 0)

        tail_start = base + NFULL * CH

        @pl.when(wid < NW - 1)
        def _():
            do_chunk(tail_start, GP)

        @pl.when(wid == NW - 1)
        def _():
            do_chunk(tail_start, TAIL_GROUPS)

    return gather_k


_gather = _make_gather_kernel()


def kernel(input_ids, table, gamma, beta):
    nt = _normalize_table(table, gamma, beta).reshape(-1)
    ids_flat = input_ids.reshape(-1).astype(jnp.int32)
    out_flat = _gather(nt, ids_flat)
    return out_flat.reshape(NB, Lseq, D)
